# baseline (speedup 1.0000x reference)
import jax
import jax.numpy as jnp
from jax.experimental import pallas as pl


def _copy_block(src_ref, out_ref):
    out_ref[...] = src_ref[...]


def kernel(last_hidden_state, input_ids):
    del input_ids
    B, S, H = last_hidden_state.shape
    ROWS = 256
    grid = (S // ROWS,)
    out = pl.pallas_call(
        _copy_block,
        grid=grid,
        in_specs=[pl.BlockSpec((1, ROWS, H), lambda i: (0, i, 0))],
        out_specs=pl.BlockSpec((1, ROWS, H), lambda i: (0, i, 0)),
        out_shape=jax.ShapeDtypeStruct((1, S, H), last_hidden_state.dtype),
    )(last_hidden_state)
    return out
